# baseline (speedup 1.0000x reference)
import jax
import jax.numpy as jnp
from jax.experimental import pallas as pl

B, C, H, W = 16, 64, 32, 32
K = 1024
BETA = 0.25
T = H * W


def _vq_body(z_ref, cb_ref, idx_ref, zq_ref, loss_ref):
    z = z_ref[0].reshape(C, T)
    zt = z.T
    cb = cb_ref[...]
    dot = jax.lax.dot_general(zt, cb, (((1,), (1,)), ((), ())),
                              preferred_element_type=jnp.float32)
    zsq = jnp.sum(zt * zt, axis=1, keepdims=True)
    cbsq = jnp.sum(cb * cb, axis=1)[None, :]
    d = zsq + cbsq - 2.0 * dot
    dmin = jnp.min(d, axis=1, keepdims=True)
    iota = jax.lax.broadcasted_iota(jnp.int32, (T, K), 1)
    idx = jnp.min(jnp.where(d == dmin, iota, K), axis=1)
    idx_ref[0, 0, :] = idx
    onehot = (iota == idx[:, None]).astype(jnp.float32)
    zq = jax.lax.dot_general(onehot, cb, (((1,), (0,)), ((), ())),
                             preferred_element_type=jnp.float32)
    zq_ref[0] = zq
    diff = zq - zt
    loss_ref[0] = jnp.sum(diff * diff, keepdims=True).reshape(1, 1)


def kernel(z_e, codebook_weight):
    idx3, zq, losspart = pl.pallas_call(
        _vq_body,
        grid=(B,),
        in_specs=[
            pl.BlockSpec((1, C, H, W), lambda b: (b, 0, 0, 0)),
            pl.BlockSpec((K, C), lambda b: (0, 0)),
        ],
        out_specs=[
            pl.BlockSpec((1, 1, T), lambda b: (b, 0, 0)),
            pl.BlockSpec((1, T, C), lambda b: (b, 0, 0)),
            pl.BlockSpec((1, 1, 1), lambda b: (b, 0, 0)),
        ],
        out_shape=[
            jax.ShapeDtypeStruct((B, 1, T), jnp.int32),
            jax.ShapeDtypeStruct((B, T, C), jnp.float32),
            jax.ShapeDtypeStruct((B, 1, 1), jnp.float32),
        ],
    )(z_e, codebook_weight)
    codebook_idx = idx3.reshape(-1, 1)
    z_q = zq.reshape(B, H, W, C).transpose(0, 3, 1, 2)
    loss_vq = jnp.sum(losspart) * ((1.0 + BETA) / (B * T * C))
    return (z_q, codebook_idx, loss_vq)
